# baseline (speedup 1.0000x reference)
import functools

import jax
import jax.numpy as jnp
from jax import lax
from jax.experimental import pallas as pl
from jax.experimental.pallas import tpu as pltpu
from jax.experimental.pallas import tpu_sc as plsc

_N = 100000
_F = 128
_NNZ = 2048
_TILE = 20000
_LANES = 16

_SC_MESH = plsc.VectorSubcoreMesh(core_axis_name="c", subcore_axis_name="s")


@functools.partial(
    pl.kernel,
    out_type=jax.ShapeDtypeStruct((_F * _F,), jnp.float32),
    mesh=_SC_MESH,
    scratch_types=[
        pltpu.VMEM((_NNZ,), jnp.int32),
        pltpu.VMEM((_NNZ,), jnp.int32),
        pltpu.VMEM((_NNZ,), jnp.float32),
        pltpu.VMEM((_NNZ,), jnp.int32),
        pltpu.VMEM_SHARED((_F * _F,), jnp.float32),
    ],
)
def _scatter_a_sc(rows_hbm, cols_hbm, vals_hbm, zeros_hbm, a_hbm,
                  rows_v, cols_v, vals_v, idx_v, acc):
    wid = lax.axis_index("s") * _SC_MESH.num_cores + lax.axis_index("c")

    @pl.when(wid == 0)
    def _():
        pltpu.sync_copy(rows_hbm, rows_v)
        pltpu.sync_copy(cols_hbm, cols_v)
        pltpu.sync_copy(vals_hbm, vals_v)
        pltpu.sync_copy(zeros_hbm, acc)

        def body(e, carry):
            r = rows_v[pl.ds(e * _LANES, _LANES)]
            c = cols_v[pl.ds(e * _LANES, _LANES)]
            idx_v[pl.ds(e * _LANES, _LANES)] = r * _F + c
            return carry

        lax.fori_loop(0, _NNZ // _LANES, body, 0)
        pltpu.sync_copy(vals_v, acc.at[idx_v], add=True)
        pltpu.sync_copy(acc, a_hbm)


def _gcn_tc_kernel(a_ref, w_ref, x_ref, o_ref, m_ref):
    @pl.when(pl.program_id(0) == 0)
    def _build_m():
        m_ref[...] = jax.lax.dot_general(
            w_ref[...], a_ref[...], (((0,), (0,)), ((), ())),
            preferred_element_type=jnp.float32)

    o_ref[...] = jnp.dot(x_ref[...], m_ref[...],
                         preferred_element_type=jnp.float32)


def kernel(x, adj_indices, adj_values, W):
    rows = adj_indices[0]
    cols = adj_indices[1]
    zeros = jnp.zeros((_F * _F,), jnp.float32)

    a = _scatter_a_sc(rows, cols, adj_values, zeros).reshape(_F, _F)

    out = pl.pallas_call(
        _gcn_tc_kernel,
        grid=(_N // _TILE,),
        in_specs=[
            pl.BlockSpec((_F, _F), lambda i: (0, 0)),
            pl.BlockSpec((_F, _F), lambda i: (0, 0)),
            pl.BlockSpec((_TILE, _F), lambda i: (i, 0)),
        ],
        out_specs=pl.BlockSpec((_TILE, _F), lambda i: (i, 0)),
        out_shape=jax.ShapeDtypeStruct((_N, _F), jnp.float32),
        scratch_shapes=[pltpu.VMEM((_F, _F), jnp.float32)],
        compiler_params=pltpu.CompilerParams(
            dimension_semantics=("arbitrary",)),
    )(a, W, x)
    return out


# speedup vs baseline: 1.5024x; 1.5024x over previous
import jax
import jax.numpy as jnp
from jax.experimental import pallas as pl
from jax.experimental.pallas import tpu as pltpu

_N = 100000
_F = 128
_NNZ = 2048
_TILE = 25000


def _gcn_kernel(rows_ref, cols_ref, vals_ref, w_ref, x_ref, o_ref, m_ref):
    @pl.when(pl.program_id(0) == 0)
    def _build_m():
        r = rows_ref[0, :]
        c = cols_ref[0, :]
        v = vals_ref[0, :]
        ids = jax.lax.broadcasted_iota(jnp.int32, (_NNZ, _F), 1)
        r_onehot = (r[:, None] == ids).astype(jnp.float32)
        cv = jnp.where(c[:, None] == ids, v[:, None], 0.0)
        a = jax.lax.dot_general(
            r_onehot, cv, (((0,), (0,)), ((), ())),
            preferred_element_type=jnp.float32)
        m_ref[...] = jax.lax.dot_general(
            w_ref[...], a, (((0,), (0,)), ((), ())),
            preferred_element_type=jnp.float32)

    o_ref[...] = jnp.dot(x_ref[...], m_ref[...],
                         preferred_element_type=jnp.float32)


def kernel(x, adj_indices, adj_values, W):
    rows = adj_indices[0].reshape(1, _NNZ)
    cols = adj_indices[1].reshape(1, _NNZ)
    vals = adj_values.reshape(1, _NNZ)

    out = pl.pallas_call(
        _gcn_kernel,
        grid=(_N // _TILE,),
        in_specs=[
            pl.BlockSpec((1, _NNZ), lambda i: (0, 0)),
            pl.BlockSpec((1, _NNZ), lambda i: (0, 0)),
            pl.BlockSpec((1, _NNZ), lambda i: (0, 0)),
            pl.BlockSpec((_F, _F), lambda i: (0, 0)),
            pl.BlockSpec((_TILE, _F), lambda i: (i, 0)),
        ],
        out_specs=pl.BlockSpec((_TILE, _F), lambda i: (i, 0)),
        out_shape=jax.ShapeDtypeStruct((_N, _F), jnp.float32),
        scratch_shapes=[pltpu.VMEM((_F, _F), jnp.float32)],
        compiler_params=pltpu.CompilerParams(
            dimension_semantics=("arbitrary",),
            vmem_limit_bytes=67108864),
    )(rows, cols, vals, W, x)
    return out


# speedup vs baseline: 1.5982x; 1.0638x over previous
import jax
import jax.numpy as jnp
from jax.experimental import pallas as pl
from jax.experimental.pallas import tpu as pltpu

_N = 100000
_F = 128
_NNZ = 2048
_TILE = 20000


def _gcn_kernel(rows_ref, cols_ref, vals_ref, w_ref, x_ref, o_ref, m_ref):
    @pl.when(pl.program_id(0) == 0)
    def _build_m():
        r = rows_ref[0, :]
        c = cols_ref[0, :]
        v = vals_ref[0, :]
        ids = jax.lax.broadcasted_iota(jnp.int32, (_NNZ, _F), 1)
        r_onehot = (r[:, None] == ids).astype(jnp.float32)
        cv = jnp.where(c[:, None] == ids, v[:, None], 0.0)
        a = jax.lax.dot_general(
            r_onehot, cv, (((0,), (0,)), ((), ())),
            preferred_element_type=jnp.float32)
        m_ref[...] = jax.lax.dot_general(
            w_ref[...], a, (((0,), (0,)), ((), ())),
            preferred_element_type=jnp.float32)

    o_ref[...] = jnp.dot(x_ref[...], m_ref[...],
                         preferred_element_type=jnp.float32)


def kernel(x, adj_indices, adj_values, W):
    rows = adj_indices[0].reshape(1, _NNZ)
    cols = adj_indices[1].reshape(1, _NNZ)
    vals = adj_values.reshape(1, _NNZ)

    out = pl.pallas_call(
        _gcn_kernel,
        grid=(_N // _TILE,),
        in_specs=[
            pl.BlockSpec((1, _NNZ), lambda i: (0, 0)),
            pl.BlockSpec((1, _NNZ), lambda i: (0, 0)),
            pl.BlockSpec((1, _NNZ), lambda i: (0, 0)),
            pl.BlockSpec((_F, _F), lambda i: (0, 0)),
            pl.BlockSpec((_TILE, _F), lambda i: (i, 0)),
        ],
        out_specs=pl.BlockSpec((_TILE, _F), lambda i: (i, 0)),
        out_shape=jax.ShapeDtypeStruct((_N, _F), jnp.float32),
        scratch_shapes=[pltpu.VMEM((_F, _F), jnp.float32)],
        compiler_params=pltpu.CompilerParams(
            dimension_semantics=("arbitrary",)),
    )(rows, cols, vals, W, x)
    return out
